# baseline (speedup 1.0000x reference)
import jax
import jax.numpy as jnp
from jax import lax
from jax.experimental import pallas as pl
from jax.experimental.pallas import tpu as pltpu
from jax.experimental.pallas import tpu_sc as plsc

NUM_CONCEPTS = 128
BATCH = 16384
MAX_SLIP = 0.4
MAX_GUESS = 0.4

_L = 16
_NW = 32
_BPW = BATCH // _NW
_CHUNK = 128
_NCHUNK = _BPW // _CHUNK


def _sc_count_kernel(uid_hbm, qid_hbm, theta_hbm, q_hbm, sg_hbm,
                     m_out, slip_out, guess_out,
                     uid_i, qid_i, sgr_i, th_v, qv_v, sg_v,
                     slip_v, guess_v, m_v,
                     sem_th, sem_q, sem_sg):
    wid = lax.axis_index("s") * 2 + lax.axis_index("c")
    base = wid * _BPW

    for c in range(_NCHUNK):
        cbase = base + c * _CHUNK
        pltpu.sync_copy(uid_hbm.at[pl.ds(cbase, _CHUNK)], uid_i)
        pltpu.sync_copy(qid_hbm.at[pl.ds(cbase, _CHUNK)], qid_i)
        for g in range(_CHUNK // _L):
            sgr_i[pl.ds(g * _L, _L)] = qid_i[pl.ds(g * _L, _L)] >> 6
        cp_th = pltpu.async_copy(theta_hbm.at[uid_i], th_v, sem_th)
        cp_q = pltpu.async_copy(q_hbm.at[qid_i], qv_v, sem_q)
        cp_sg = pltpu.async_copy(sg_hbm.at[sgr_i], sg_v, sem_sg)
        cp_th.wait()
        cp_q.wait()
        cp_sg.wait()

        def body(g, carry):
            rows = g * _L + lax.iota(jnp.int32, _L)
            acc = jnp.zeros((_L,), jnp.float32)
            for k in range(NUM_CONCEPTS):
                cols = jnp.full((_L,), k, jnp.int32)
                th = plsc.load_gather(th_v, [rows, cols])
                qk = plsc.load_gather(qv_v, [rows, cols])
                bad = (qk == 1) & (th <= 0.0)
                acc = acc + jnp.where(bad, 1.0, 0.0)
            qid_g = plsc.load_gather(qid_i, [rows])
            scol = (qid_g & 63) << 1
            sl = plsc.load_gather(sg_v, [rows, scol])
            gu = plsc.load_gather(sg_v, [rows, scol + 1])
            off = pl.multiple_of(c * _CHUNK + g * _L, _L)
            m_v[pl.ds(off, _L)] = acc
            slip_v[pl.ds(off, _L)] = sl
            guess_v[pl.ds(off, _L)] = gu
            return carry

        lax.fori_loop(0, _CHUNK // _L, body, 0)

    pltpu.sync_copy(m_v, m_out.at[pl.ds(base, _BPW)])
    pltpu.sync_copy(slip_v, slip_out.at[pl.ds(base, _BPW)])
    pltpu.sync_copy(guess_v, guess_out.at[pl.ds(base, _BPW)])


@jax.jit
def _sc_counts(uid, qid, theta_table, q_table, sg_table):
    mesh = plsc.VectorSubcoreMesh(core_axis_name="c", subcore_axis_name="s")
    f = pl.kernel(
        _sc_count_kernel,
        mesh=mesh,
        compiler_params=pltpu.CompilerParams(needs_layout_passes=False),
        out_type=[
            jax.ShapeDtypeStruct((BATCH,), jnp.float32),
            jax.ShapeDtypeStruct((BATCH,), jnp.float32),
            jax.ShapeDtypeStruct((BATCH,), jnp.float32),
        ],
        scratch_types=[
            pltpu.VMEM((_CHUNK,), jnp.int32),
            pltpu.VMEM((_CHUNK,), jnp.int32),
            pltpu.VMEM((_CHUNK,), jnp.int32),
            pltpu.VMEM((_CHUNK, NUM_CONCEPTS), jnp.float32),
            pltpu.VMEM((_CHUNK, NUM_CONCEPTS), jnp.int32),
            pltpu.VMEM((_CHUNK, NUM_CONCEPTS), jnp.float32),
            pltpu.VMEM((_BPW,), jnp.float32),
            pltpu.VMEM((_BPW,), jnp.float32),
            pltpu.VMEM((_BPW,), jnp.float32),
            pltpu.SemaphoreType.DMA,
            pltpu.SemaphoreType.DMA,
            pltpu.SemaphoreType.DMA,
        ],
    )
    return f(uid, qid, theta_table, q_table, sg_table)


def _tc_combine_kernel(m_ref, s_ref, g_ref, o_ref):
    m = m_ref[...]
    slip = jax.nn.sigmoid(s_ref[...]) * MAX_SLIP
    guess = jax.nn.sigmoid(g_ref[...]) * MAX_GUESS
    n = jnp.exp(m * (-0.6931471805599453))
    o_ref[...] = jnp.exp(n * jnp.log(1.0 - slip) + (1.0 - n) * jnp.log(guess))


def kernel(user_id, question_id, theta_table, slip_table, guess_table, q_table):
    uid = user_id.astype(jnp.int32)
    qid = question_id.astype(jnp.int32)
    nq = slip_table.shape[0]
    nrows = (nq * 2 + 127) // 128
    sg = jnp.concatenate([slip_table, guess_table], axis=1).reshape(-1)
    sg = jnp.pad(sg, (0, nrows * 128 - 2 * nq)).reshape(nrows, 128)
    m, slip_raw, guess_raw = _sc_counts(uid, qid, theta_table, q_table, sg)
    out = pl.pallas_call(
        _tc_combine_kernel,
        out_shape=jax.ShapeDtypeStruct((128, 128), jnp.float32),
    )(m.reshape(128, 128), slip_raw.reshape(128, 128), guess_raw.reshape(128, 128))
    return out.reshape(BATCH)


# speedup vs baseline: 1.2164x; 1.2164x over previous
import jax
import jax.numpy as jnp
from jax import lax
from jax.experimental import pallas as pl
from jax.experimental.pallas import tpu as pltpu
from jax.experimental.pallas import tpu_sc as plsc

NUM_CONCEPTS = 128
BATCH = 16384
MAX_SLIP = 0.4
MAX_GUESS = 0.4

_L = 16
_NW = 32
_BPW = BATCH // _NW
_CHUNK = 128
_NCHUNK = _BPW // _CHUNK


def _sc_gather_kernel(uid_hbm, qid_hbm, theta_hbm, q_hbm, sg_hbm,
                      th_out, q_out, slip_out, guess_out,
                      uid_i0, uid_i1, qid_i0, qid_i1, sgr_i,
                      th_v0, th_v1, qv_v0, qv_v1, sg_v,
                      slip_v, guess_v,
                      sem_th0, sem_th1, sem_q0, sem_q1, sem_sg,
                      sem_wth0, sem_wth1, sem_wq0, sem_wq1):
    wid = lax.axis_index("s") * 2 + lax.axis_index("c")
    base = wid * _BPW

    uid_bufs = (uid_i0, uid_i1)
    qid_bufs = (qid_i0, qid_i1)
    th_bufs = (th_v0, th_v1)
    qv_bufs = (qv_v0, qv_v1)
    gsems = ((sem_th0, sem_q0), (sem_th1, sem_q1))
    wsems = ((sem_wth0, sem_wq0), (sem_wth1, sem_wq1))

    gathers = [None, None]
    writes = [None, None]

    def start_chunk(c, slot):
        if writes[slot] is not None:
            writes[slot][0].wait()
            writes[slot][1].wait()
            writes[slot] = None
        cbase = base + c * _CHUNK
        pltpu.sync_copy(uid_hbm.at[pl.ds(cbase, _CHUNK)], uid_bufs[slot])
        pltpu.sync_copy(qid_hbm.at[pl.ds(cbase, _CHUNK)], qid_bufs[slot])
        cp_th = pltpu.async_copy(theta_hbm.at[uid_bufs[slot]],
                                 th_bufs[slot], gsems[slot][0])
        cp_q = pltpu.async_copy(q_hbm.at[qid_bufs[slot]],
                                qv_bufs[slot], gsems[slot][1])
        gathers[slot] = (cp_th, cp_q)

    start_chunk(0, 0)

    for c in range(_NCHUNK):
        slot = c & 1
        cbase = base + c * _CHUNK
        if c + 1 < _NCHUNK:
            start_chunk(c + 1, (c + 1) & 1)
        gathers[slot][0].wait()
        gathers[slot][1].wait()

        qid_i = qid_bufs[slot]
        for g in range(_CHUNK // _L):
            sgr_i[pl.ds(g * _L, _L)] = qid_i[pl.ds(g * _L, _L)] >> 6
        cp_sg = pltpu.async_copy(sg_hbm.at[sgr_i], sg_v, sem_sg)
        cp_sg.wait()

        def body(g, carry):
            rows = g * _L + lax.iota(jnp.int32, _L)
            qid_g = plsc.load_gather(qid_i, [rows])
            scol = (qid_g & 63) << 1
            sl = plsc.load_gather(sg_v, [rows, scol])
            gu = plsc.load_gather(sg_v, [rows, scol + 1])
            off = pl.multiple_of(c * _CHUNK + g * _L, _L)
            slip_v[pl.ds(off, _L)] = sl
            guess_v[pl.ds(off, _L)] = gu
            return carry

        lax.fori_loop(0, _CHUNK // _L, body, 0)

        w_th = pltpu.async_copy(th_bufs[slot],
                                th_out.at[pl.ds(cbase, _CHUNK)], wsems[slot][0])
        w_q = pltpu.async_copy(qv_bufs[slot],
                               q_out.at[pl.ds(cbase, _CHUNK)], wsems[slot][1])
        writes[slot] = (w_th, w_q)

    for w in writes:
        if w is not None:
            w[0].wait()
            w[1].wait()

    pltpu.sync_copy(slip_v, slip_out.at[pl.ds(base, _BPW)])
    pltpu.sync_copy(guess_v, guess_out.at[pl.ds(base, _BPW)])


@jax.jit
def _sc_gather(uid, qid, theta_table, q_table, sg_table):
    mesh = plsc.VectorSubcoreMesh(core_axis_name="c", subcore_axis_name="s")
    f = pl.kernel(
        _sc_gather_kernel,
        mesh=mesh,
        compiler_params=pltpu.CompilerParams(needs_layout_passes=False),
        out_type=[
            jax.ShapeDtypeStruct((BATCH, NUM_CONCEPTS), jnp.float32),
            jax.ShapeDtypeStruct((BATCH, NUM_CONCEPTS), jnp.int32),
            jax.ShapeDtypeStruct((BATCH,), jnp.float32),
            jax.ShapeDtypeStruct((BATCH,), jnp.float32),
        ],
        scratch_types=[
            pltpu.VMEM((_CHUNK,), jnp.int32),
            pltpu.VMEM((_CHUNK,), jnp.int32),
            pltpu.VMEM((_CHUNK,), jnp.int32),
            pltpu.VMEM((_CHUNK,), jnp.int32),
            pltpu.VMEM((_CHUNK,), jnp.int32),
            pltpu.VMEM((_CHUNK, NUM_CONCEPTS), jnp.float32),
            pltpu.VMEM((_CHUNK, NUM_CONCEPTS), jnp.float32),
            pltpu.VMEM((_CHUNK, NUM_CONCEPTS), jnp.int32),
            pltpu.VMEM((_CHUNK, NUM_CONCEPTS), jnp.int32),
            pltpu.VMEM((_CHUNK, NUM_CONCEPTS), jnp.float32),
            pltpu.VMEM((_BPW,), jnp.float32),
            pltpu.VMEM((_BPW,), jnp.float32),
            pltpu.SemaphoreType.DMA,
            pltpu.SemaphoreType.DMA,
            pltpu.SemaphoreType.DMA,
            pltpu.SemaphoreType.DMA,
            pltpu.SemaphoreType.DMA,
            pltpu.SemaphoreType.DMA,
            pltpu.SemaphoreType.DMA,
            pltpu.SemaphoreType.DMA,
            pltpu.SemaphoreType.DMA,
        ],
    )
    return f(uid, qid, theta_table, q_table, sg_table)


_TC_BLOCK = 1024
_TC_GRID = BATCH // _TC_BLOCK


def _tc_combine_kernel(th_ref, q_ref, s_ref, g_ref, o_ref):
    bad = (q_ref[...] == 1) & (th_ref[...] <= 0.0)
    m = jnp.sum(bad.astype(jnp.float32), axis=1, keepdims=True)
    slip = jax.nn.sigmoid(s_ref[...]) * MAX_SLIP
    guess = jax.nn.sigmoid(g_ref[...]) * MAX_GUESS
    n = jnp.exp(m * (-0.6931471805599453))
    o_ref[...] = jnp.exp(n * jnp.log(1.0 - slip) + (1.0 - n) * jnp.log(guess))


def kernel(user_id, question_id, theta_table, slip_table, guess_table, q_table):
    uid = user_id.astype(jnp.int32)
    qid = question_id.astype(jnp.int32)
    nq = slip_table.shape[0]
    nrows = (nq * 2 + 127) // 128
    sg = jnp.concatenate([slip_table, guess_table], axis=1).reshape(-1)
    sg = jnp.pad(sg, (0, nrows * 128 - 2 * nq)).reshape(nrows, 128)
    th_g, q_g, slip_raw, guess_raw = _sc_gather(uid, qid, theta_table,
                                                q_table, sg)
    out = pl.pallas_call(
        _tc_combine_kernel,
        grid=(_TC_GRID,),
        in_specs=[
            pl.BlockSpec((_TC_BLOCK, NUM_CONCEPTS), lambda i: (i, 0)),
            pl.BlockSpec((_TC_BLOCK, NUM_CONCEPTS), lambda i: (i, 0)),
            pl.BlockSpec((_TC_BLOCK, 1), lambda i: (i, 0)),
            pl.BlockSpec((_TC_BLOCK, 1), lambda i: (i, 0)),
        ],
        out_specs=pl.BlockSpec((_TC_BLOCK, 1), lambda i: (i, 0)),
        out_shape=jax.ShapeDtypeStruct((BATCH, 1), jnp.float32),
    )(th_g, q_g, slip_raw.reshape(BATCH, 1), guess_raw.reshape(BATCH, 1))
    return out.reshape(BATCH)


# speedup vs baseline: 1.9670x; 1.6171x over previous
import jax
import jax.numpy as jnp
from jax import lax
from jax.experimental import pallas as pl
from jax.experimental.pallas import tpu as pltpu
from jax.experimental.pallas import tpu_sc as plsc

NUM_CONCEPTS = 128
BATCH = 16384
MAX_SLIP = 0.4
MAX_GUESS = 0.4

_L = 16
_NW = 32
_BPW = BATCH // _NW
_CHUNK = 128
_NCHUNK = _BPW // _CHUNK


def _sc_gather_kernel(uid_hbm, qid_hbm, theta_hbm, q_hbm, sl2_hbm, gu2_hbm,
                      th_out, q_out, slip_out, guess_out,
                      uid_i0, uid_i1, qid_i0, qid_i1, sr_i,
                      th_v0, th_v1, qv_v0, qv_v1, slr_v, gur_v,
                      slip_v, guess_v,
                      sem_g0, sem_g1, sem_sg, sem_w0, sem_w1):
    wid = lax.axis_index("s") * 2 + lax.axis_index("c")
    base = wid * _BPW

    uid_bufs = (uid_i0, uid_i1)
    qid_bufs = (qid_i0, qid_i1)
    th_bufs = (th_v0, th_v1)
    qv_bufs = (qv_v0, qv_v1)
    gsems = (sem_g0, sem_g1)
    wsems = (sem_w0, sem_w1)

    gathers = [None, None]
    writes = [None, None]

    def start_chunk(c, slot):
        if writes[slot] is not None:
            for w in writes[slot]:
                w.wait()
            writes[slot] = None
        cbase = base + c * _CHUNK
        pltpu.sync_copy(uid_hbm.at[pl.ds(cbase, _CHUNK)], uid_bufs[slot])
        pltpu.sync_copy(qid_hbm.at[pl.ds(cbase, _CHUNK)], qid_bufs[slot])
        gathers[slot] = (
            pltpu.async_copy(theta_hbm.at[uid_bufs[slot]], th_bufs[slot],
                             gsems[slot]),
            pltpu.async_copy(q_hbm.at[qid_bufs[slot]], qv_bufs[slot],
                             gsems[slot]),
        )

    start_chunk(0, 0)

    for c in range(_NCHUNK):
        slot = c & 1
        cbase = base + c * _CHUNK
        if c + 1 < _NCHUNK:
            start_chunk(c + 1, (c + 1) & 1)

        qid_i = qid_bufs[slot]
        for g in range(_CHUNK // _L):
            sr_i[pl.ds(g * _L, _L)] = qid_i[pl.ds(g * _L, _L)] >> 7
        cp_sl = pltpu.async_copy(sl2_hbm.at[sr_i], slr_v, sem_sg)
        cp_gu = pltpu.async_copy(gu2_hbm.at[sr_i], gur_v, sem_sg)

        for g in gathers[slot]:
            g.wait()
        cp_sl.wait()
        cp_gu.wait()

        def body(g, carry):
            rows = g * _L + lax.iota(jnp.int32, _L)
            qid_g = plsc.load_gather(qid_i, [rows])
            scol = qid_g & 127
            sl = plsc.load_gather(slr_v, [rows, scol])
            gu = plsc.load_gather(gur_v, [rows, scol])
            off = pl.multiple_of(c * _CHUNK + g * _L, _L)
            slip_v[pl.ds(off, _L)] = sl
            guess_v[pl.ds(off, _L)] = gu
            return carry

        lax.fori_loop(0, _CHUNK // _L, body, 0)

        writes[slot] = (
            pltpu.async_copy(th_bufs[slot], th_out.at[pl.ds(cbase, _CHUNK)],
                             wsems[slot]),
            pltpu.async_copy(qv_bufs[slot], q_out.at[pl.ds(cbase, _CHUNK)],
                             wsems[slot]),
        )

    for ws in writes:
        if ws is not None:
            for w in ws:
                w.wait()

    pltpu.sync_copy(slip_v, slip_out.at[pl.ds(base, _BPW)])
    pltpu.sync_copy(guess_v, guess_out.at[pl.ds(base, _BPW)])


@jax.jit
def _sc_gather(uid, qid, theta_table, q_table, sl2, gu2):
    mesh = plsc.VectorSubcoreMesh(core_axis_name="c", subcore_axis_name="s")
    f = pl.kernel(
        _sc_gather_kernel,
        mesh=mesh,
        compiler_params=pltpu.CompilerParams(needs_layout_passes=False),
        out_type=[
            jax.ShapeDtypeStruct((BATCH, NUM_CONCEPTS), jnp.float32),
            jax.ShapeDtypeStruct((BATCH, NUM_CONCEPTS), jnp.int32),
            jax.ShapeDtypeStruct((BATCH,), jnp.float32),
            jax.ShapeDtypeStruct((BATCH,), jnp.float32),
        ],
        scratch_types=[
            pltpu.VMEM((_CHUNK,), jnp.int32),
            pltpu.VMEM((_CHUNK,), jnp.int32),
            pltpu.VMEM((_CHUNK,), jnp.int32),
            pltpu.VMEM((_CHUNK,), jnp.int32),
            pltpu.VMEM((_CHUNK,), jnp.int32),
            pltpu.VMEM((_CHUNK, NUM_CONCEPTS), jnp.float32),
            pltpu.VMEM((_CHUNK, NUM_CONCEPTS), jnp.float32),
            pltpu.VMEM((_CHUNK, NUM_CONCEPTS), jnp.int32),
            pltpu.VMEM((_CHUNK, NUM_CONCEPTS), jnp.int32),
            pltpu.VMEM((_CHUNK, NUM_CONCEPTS), jnp.float32),
            pltpu.VMEM((_CHUNK, NUM_CONCEPTS), jnp.float32),
            pltpu.VMEM((_BPW,), jnp.float32),
            pltpu.VMEM((_BPW,), jnp.float32),
            pltpu.SemaphoreType.DMA,
            pltpu.SemaphoreType.DMA,
            pltpu.SemaphoreType.DMA,
            pltpu.SemaphoreType.DMA,
            pltpu.SemaphoreType.DMA,
        ],
    )
    return f(uid, qid, theta_table, q_table, sl2, gu2)


_TC_BLOCK = 1024
_TC_GRID = BATCH // _TC_BLOCK


def _tc_combine_kernel(th_ref, q_ref, s_ref, g_ref, o_ref):
    bad = (q_ref[...] == 1) & (th_ref[...] <= 0.0)
    m = jnp.sum(bad.astype(jnp.float32), axis=1, keepdims=True)
    slip = jax.nn.sigmoid(s_ref[...]) * MAX_SLIP
    guess = jax.nn.sigmoid(g_ref[...]) * MAX_GUESS
    n = jnp.exp(m * (-0.6931471805599453))
    o_ref[...] = jnp.exp(n * jnp.log(1.0 - slip) + (1.0 - n) * jnp.log(guess))


def kernel(user_id, question_id, theta_table, slip_table, guess_table, q_table):
    uid = user_id.astype(jnp.int32)
    qid = question_id.astype(jnp.int32)
    nq = slip_table.shape[0]
    nrows = (nq + 127) // 128
    pad = nrows * 128 - nq
    sl2 = jnp.pad(slip_table.reshape(nq), (0, pad)).reshape(nrows, 128)
    gu2 = jnp.pad(guess_table.reshape(nq), (0, pad)).reshape(nrows, 128)
    th_g, q_g, slip_raw, guess_raw = _sc_gather(uid, qid, theta_table,
                                                q_table, sl2, gu2)
    out = pl.pallas_call(
        _tc_combine_kernel,
        grid=(_TC_GRID,),
        in_specs=[
            pl.BlockSpec((_TC_BLOCK, NUM_CONCEPTS), lambda i: (i, 0)),
            pl.BlockSpec((_TC_BLOCK, NUM_CONCEPTS), lambda i: (i, 0)),
            pl.BlockSpec((_TC_BLOCK, 1), lambda i: (i, 0)),
            pl.BlockSpec((_TC_BLOCK, 1), lambda i: (i, 0)),
        ],
        out_specs=pl.BlockSpec((_TC_BLOCK, 1), lambda i: (i, 0)),
        out_shape=jax.ShapeDtypeStruct((BATCH, 1), jnp.float32),
    )(th_g, q_g, slip_raw.reshape(BATCH, 1), guess_raw.reshape(BATCH, 1))
    return out.reshape(BATCH)


# speedup vs baseline: 3.1827x; 1.6181x over previous
import jax
import jax.numpy as jnp
from jax import lax
from jax.experimental import pallas as pl
from jax.experimental.pallas import tpu as pltpu
from jax.experimental.pallas import tpu_sc as plsc

NUM_CONCEPTS = 128
BATCH = 16384
MAX_SLIP = 0.4
MAX_GUESS = 0.4

_L = 16
_NW = 32
_BPW = BATCH // _NW
_CHUNK = 128
_NCHUNK = _BPW // _CHUNK


def _sc_count_kernel(uid_hbm, qid_hbm, theta_hbm, q_hbm, sl2_hbm, gu2_hbm,
                     m_out, slip_out, guess_out,
                     uid_i0, uid_i1, qid_i0, qid_i1, sr_i,
                     th_v0, th_v1, qv_v0, qv_v1, slr_v, gur_v,
                     m_v, slip_v, guess_v,
                     sem_g0, sem_g1, sem_sg):
    wid = lax.axis_index("s") * 2 + lax.axis_index("c")
    base = wid * _BPW

    uid_bufs = (uid_i0, uid_i1)
    qid_bufs = (qid_i0, qid_i1)
    th_bufs = (th_v0, th_v1)
    qv_bufs = (qv_v0, qv_v1)
    gsems = (sem_g0, sem_g1)

    gathers = [None, None]

    def start_chunk(c, slot):
        cbase = base + c * _CHUNK
        pltpu.sync_copy(uid_hbm.at[pl.ds(cbase, _CHUNK)], uid_bufs[slot])
        pltpu.sync_copy(qid_hbm.at[pl.ds(cbase, _CHUNK)], qid_bufs[slot])
        gathers[slot] = (
            pltpu.async_copy(theta_hbm.at[uid_bufs[slot]], th_bufs[slot],
                             gsems[slot]),
            pltpu.async_copy(q_hbm.at[qid_bufs[slot]], qv_bufs[slot],
                             gsems[slot]),
        )

    start_chunk(0, 0)

    for c in range(_NCHUNK):
        slot = c & 1
        if c + 1 < _NCHUNK:
            start_chunk(c + 1, (c + 1) & 1)

        qid_i = qid_bufs[slot]
        for g in range(_CHUNK // _L):
            sr_i[pl.ds(g * _L, _L)] = qid_i[pl.ds(g * _L, _L)] >> 7
        cp_sl = pltpu.async_copy(sl2_hbm.at[sr_i], slr_v, sem_sg)
        cp_gu = pltpu.async_copy(gu2_hbm.at[sr_i], gur_v, sem_sg)

        for g in gathers[slot]:
            g.wait()
        cp_sl.wait()
        cp_gu.wait()

        th_v = th_bufs[slot]
        qv_v = qv_bufs[slot]

        lanes = lax.iota(jnp.int32, _L)
        for g in range(_CHUNK // _L):

            def srow(r, rsum):
                row = g * _L + r
                acc = jnp.zeros((_L,), jnp.float32)
                for j in range(NUM_CONCEPTS // _L):
                    th = th_v[row, pl.ds(j * _L, _L)]
                    qv = qv_v[row, pl.ds(j * _L, _L)]
                    acc = acc + jnp.where((qv == 1) & (th <= 0.0), 1.0, 0.0)
                return jnp.where(lanes == r, jnp.sum(acc), rsum)

            rsum = lax.fori_loop(0, _L, srow, jnp.zeros((_L,), jnp.float32))
            m_v[pl.ds(c * _CHUNK + g * _L, _L)] = rsum

        def body(g, carry):
            rows = g * _L + lax.iota(jnp.int32, _L)
            qid_g = plsc.load_gather(qid_i, [rows])
            scol = qid_g & 127
            sl = plsc.load_gather(slr_v, [rows, scol])
            gu = plsc.load_gather(gur_v, [rows, scol])
            off = pl.multiple_of(c * _CHUNK + g * _L, _L)
            slip_v[pl.ds(off, _L)] = sl
            guess_v[pl.ds(off, _L)] = gu
            return carry

        lax.fori_loop(0, _CHUNK // _L, body, 0)

    pltpu.sync_copy(m_v, m_out.at[pl.ds(base, _BPW)])
    pltpu.sync_copy(slip_v, slip_out.at[pl.ds(base, _BPW)])
    pltpu.sync_copy(guess_v, guess_out.at[pl.ds(base, _BPW)])


@jax.jit
def _sc_count(uid, qid, theta_table, q_table, sl2, gu2):
    mesh = plsc.VectorSubcoreMesh(core_axis_name="c", subcore_axis_name="s")
    f = pl.kernel(
        _sc_count_kernel,
        mesh=mesh,
        compiler_params=pltpu.CompilerParams(needs_layout_passes=False),
        out_type=[
            jax.ShapeDtypeStruct((BATCH,), jnp.float32),
            jax.ShapeDtypeStruct((BATCH,), jnp.float32),
            jax.ShapeDtypeStruct((BATCH,), jnp.float32),
        ],
        scratch_types=[
            pltpu.VMEM((_CHUNK,), jnp.int32),
            pltpu.VMEM((_CHUNK,), jnp.int32),
            pltpu.VMEM((_CHUNK,), jnp.int32),
            pltpu.VMEM((_CHUNK,), jnp.int32),
            pltpu.VMEM((_CHUNK,), jnp.int32),
            pltpu.VMEM((_CHUNK, NUM_CONCEPTS), jnp.float32),
            pltpu.VMEM((_CHUNK, NUM_CONCEPTS), jnp.float32),
            pltpu.VMEM((_CHUNK, NUM_CONCEPTS), jnp.int32),
            pltpu.VMEM((_CHUNK, NUM_CONCEPTS), jnp.int32),
            pltpu.VMEM((_CHUNK, NUM_CONCEPTS), jnp.float32),
            pltpu.VMEM((_CHUNK, NUM_CONCEPTS), jnp.float32),
            pltpu.VMEM((_BPW,), jnp.float32),
            pltpu.VMEM((_BPW,), jnp.float32),
            pltpu.VMEM((_BPW,), jnp.float32),
            pltpu.SemaphoreType.DMA,
            pltpu.SemaphoreType.DMA,
            pltpu.SemaphoreType.DMA,
        ],
    )
    return f(uid, qid, theta_table, q_table, sl2, gu2)


def _tc_combine_kernel(m_ref, s_ref, g_ref, o_ref):
    slip = jax.nn.sigmoid(s_ref[...]) * MAX_SLIP
    guess = jax.nn.sigmoid(g_ref[...]) * MAX_GUESS
    n = jnp.exp(m_ref[...] * (-0.6931471805599453))
    o_ref[...] = jnp.exp(n * jnp.log(1.0 - slip) + (1.0 - n) * jnp.log(guess))


def kernel(user_id, question_id, theta_table, slip_table, guess_table, q_table):
    uid = user_id.astype(jnp.int32)
    qid = question_id.astype(jnp.int32)
    nq = slip_table.shape[0]
    nrows = (nq + 127) // 128
    pad = nrows * 128 - nq
    sl2 = jnp.pad(slip_table.reshape(nq), (0, pad)).reshape(nrows, 128)
    gu2 = jnp.pad(guess_table.reshape(nq), (0, pad)).reshape(nrows, 128)
    m_raw, slip_raw, guess_raw = _sc_count(uid, qid, theta_table,
                                           q_table, sl2, gu2)
    out = pl.pallas_call(
        _tc_combine_kernel,
        out_shape=jax.ShapeDtypeStruct((128, 128), jnp.float32),
    )(m_raw.reshape(128, 128), slip_raw.reshape(128, 128),
      guess_raw.reshape(128, 128))
    return out.reshape(BATCH)
